# baseline (speedup 1.0000x reference)
import functools

import jax
import jax.numpy as jnp
from jax import lax
from jax.experimental import pallas as pl
from jax.experimental.pallas import tpu as pltpu
from jax.experimental.pallas import tpu_sc as plsc

L = 16
NC = 2
NS = 16
NW = NC * NS
B, H, W = 16, 512, 512
NROWS = B * H
TS = 1000
ROWW = 1024
TRASH_X = TS
TRASH_Y = 1008
HOLW = L * ROWW
GROUPS = NROWS // L
G_PER_W = GROUPS // NW
NACC = 9
UNROLL = 8
CHX = 8
CHY = 4
PHANTOM = 8.0 * NROWS


def _q16(x, trash):
    q0 = (x * jnp.float32(TS)).astype(jnp.int32) - 1
    q = jnp.where(q0 < 0, q0 + TS, q0)
    return jnp.where(x == 0.0, trash, q)


def _make_kernel():
    mesh = plsc.VectorSubcoreMesh(core_axis_name="c", subcore_axis_name="s")

    @functools.partial(
        pl.kernel,
        out_type=jax.ShapeDtypeStruct((NW, NACC * L), jnp.float32),
        mesh=mesh,
        compiler_params=pltpu.CompilerParams(
            use_tc_tiling_on_sc=False, needs_layout_passes=False),
        scratch_types=[
            pltpu.VMEM((H, L), jnp.float32),
            pltpu.VMEM((H, L), jnp.float32),
            pltpu.VMEM((H, L), jnp.float32),
            pltpu.VMEM((H, L), jnp.float32),
            pltpu.VMEM((L, W), jnp.float32),
            pltpu.VMEM((L, W), jnp.float32),
            pltpu.VMEM((L, W), jnp.float32),
            pltpu.VMEM((L, W), jnp.float32),
            pltpu.VMEM((HOLW,), jnp.float32),
            pltpu.VMEM((HOLW,), jnp.float32),
            pltpu.VMEM((NACC * L,), jnp.float32),
            pltpu.SemaphoreType.DMA,
            pltpu.SemaphoreType.DMA,
            pltpu.SemaphoreType.DMA,
            pltpu.SemaphoreType.DMA,
            pltpu.SemaphoreType.DMA,
            pltpu.SemaphoreType.DMA,
            pltpu.SemaphoreType.DMA,
            pltpu.SemaphoreType.DMA,
        ],
    )
    def acb_sc(cat_hbm, out_hbm,
               xr0, xt0, xr1, xt1, yr0, yt0, yr1, yt1, hr, ht, ostage,
               sxr0, sxt0, sxr1, sxt1, syr0, syt0, syr1, syt1):
        xbufs = ((xr0, xt0, sxr0, sxt0), (xr1, xt1, sxr1, sxt1))
        ybufs = ((yr0, yt0, syr0, syt0), (yr1, yt1, syr1, syt1))
        wid = lax.axis_index("s") * NC + lax.axis_index("c")
        iot = lax.iota(jnp.int32, L)
        iotf = iot.astype(jnp.float32)
        perm = jnp.minimum(iot + 1, L - 1)
        last_lane = iot == (L - 1)
        zero16 = jnp.zeros((L,), jnp.float32)

        def zinit(k, _):
            hr[pl.ds(k * L, L)] = zero16
            ht[pl.ds(k * L, L)] = zero16
            return 0
        lax.fori_loop(0, HOLW // L, zinit, 0, unroll=UNROLL)

        def stats(r, t, a0, a1, c0):
            d = r - t
            dd = d * d
            z = t == 0.0
            return (a0 + jnp.where(z, dd, 0.0),
                    a1 + jnp.where(z, 0.0, dd),
                    c0 + jnp.where(z, 1.0, 0.0))

        def scan_chunk(off, accs):
            a0, asq, c0i = accs
            r = hr[pl.ds(off, L)]
            t = ht[pl.ds(off, L)]
            hr[pl.ds(off, L)] = zero16
            ht[pl.ds(off, L)] = zero16
            d = r - t
            dd = d * d
            z = t == 0.0
            return (a0 + jnp.where(z, dd, 0.0), asq + dd,
                    c0i + plsc.all_reduce_population_count(z))

        def x_start(i, buf):
            rbuf, tbuf, rsem, tsem = buf
            g = wid * G_PER_W + i
            b = g // (W // L)
            m0 = (g % (W // L)) * L
            src = lambda o: cat_hbm.at[pl.ds(o + b * H, H), pl.ds(m0, L)]
            pltpu.async_copy(src(0), rbuf, rsem)
            pltpu.async_copy(src(NROWS), tbuf, tsem)

        def x_wait(buf):
            rbuf, tbuf, rsem, tsem = buf
            dummy = cat_hbm.at[pl.ds(0, H), pl.ds(0, L)]
            pltpu.make_async_copy(dummy, rbuf, rsem).wait()
            pltpu.make_async_copy(dummy, tbuf, tsem).wait()

        def x_group(buf):
            rbuf, tbuf = buf[0], buf[1]

            def chunk(i, _):
                idxs, vals = [], []
                for k in range(CHX):
                    s = i * CHX + k
                    sb = jnp.broadcast_to(s, (L,))
                    r = plsc.load_gather(rbuf, [sb, iot])
                    t = plsc.load_gather(tbuf, [sb, iot])
                    idxs.append((_q16(r, TRASH_X) * L + iot,
                                 _q16(t, TRASH_X) * L + iot))
                    vals.append(jnp.broadcast_to(s.astype(jnp.float32), (L,)))
                for (qr, qt), sf in zip(idxs, vals):
                    plsc.store_scatter(hr, [qr], sf)
                    plsc.store_scatter(ht, [qt], sf)
                return 0

            lax.fori_loop(0, H // CHX, chunk, 0)

        def x_scan(accs):
            def body(k, accs):
                return scan_chunk(k * L, accs)
            accs = lax.fori_loop(0, TS, body, accs, unroll=UNROLL)
            hr[pl.ds(TRASH_X * L, L)] = zero16
            ht[pl.ds(TRASH_X * L, L)] = zero16
            return accs

        def y_start(i, buf):
            rbuf, tbuf, rsem, tsem = buf
            g = wid * G_PER_W + i
            src = lambda o: cat_hbm.at[pl.ds(o + g * L, L), :]
            pltpu.async_copy(src(0), rbuf, rsem)
            pltpu.async_copy(src(NROWS), tbuf, tsem)

        def y_wait(buf):
            rbuf, tbuf, rsem, tsem = buf
            dummy = cat_hbm.at[pl.ds(0, L), :]
            pltpu.make_async_copy(dummy, rbuf, rsem).wait()
            pltpu.make_async_copy(dummy, tbuf, tsem).wait()

        def winners(x, cf, trash):
            q = _q16(x, trash)
            key = (q << 4) | iot
            sk, sv = plsc.sort_key_val(key, cf)
            qs = lax.shift_right_logical(sk, 4)
            nxt = qs.at[perm].get(mode="promise_in_bounds")
            keep = (qs != nxt) | last_lane
            return qs, sv, keep

        def y_group(buf, accs):
            rbuf, tbuf = buf[0], buf[1]

            def row(l, accs):
                lb = jnp.broadcast_to(l, (L,))
                hb = jnp.broadcast_to(l * ROWW, (L,))

                def chunk(jc, accs):
                    pend = []
                    for k in range(CHY):
                        j = jc * CHY + k
                        cb = jnp.broadcast_to(j * L, (L,)) + iot
                        r = plsc.load_gather(rbuf, [lb, cb])
                        t = plsc.load_gather(tbuf, [lb, cb])
                        accs = stats(r, t, *accs)
                        cf = cb.astype(jnp.float32)
                        pend.append(winners(r, cf, TRASH_Y)
                                    + winners(t, cf, TRASH_Y))
                    for qr, vr, kr, qt, vt_, kt_ in pend:
                        plsc.store_scatter(hr, [hb + qr], vr, mask=kr)
                        plsc.store_scatter(ht, [hb + qt], vt_, mask=kt_)
                    return accs

                return lax.fori_loop(0, W // L // CHY, chunk, accs)

            return lax.fori_loop(0, L, row, accs)

        def y_scan(accs):
            def row(l, accs):
                base = l * ROWW

                def body(k, accs):
                    return scan_chunk(base + k * L, accs)
                accs = lax.fori_loop(0, 63, body, accs, unroll=UNROLL)
                hr[pl.ds(base + TRASH_Y, L)] = zero16
                ht[pl.ds(base + TRASH_Y, L)] = zero16
                return accs

            return lax.fori_loop(0, L, row, accs)

        def run_pass(bufs, start, wait, group, accs):
            start(0, bufs[0])

            def body(k, accs):
                for parity in (0, 1):
                    i = 2 * k + parity
                    wait(bufs[parity])
                    start(jnp.minimum(i + 1, G_PER_W - 1), bufs[1 - parity])
                    accs = group(bufs[parity], accs)
                return accs

            accs = lax.fori_loop(0, G_PER_W // 2, body, accs)
            wait(bufs[0])
            return accs

        z = zero16
        zi = jnp.zeros((L,), jnp.int32)

        def xg(buf, accs):
            x_group(buf)
            return x_scan(accs)

        ax0, axsq, cx0i = run_pass(xbufs, x_start, x_wait, xg, (z, z, zi))

        def yg(buf, accs):
            vt0, vt1, vtc, a0, asq, c0i = accs
            vt0, vt1, vtc = y_group(buf, (vt0, vt1, vtc))
            a0, asq, c0i = y_scan((a0, asq, c0i))
            return vt0, vt1, vtc, a0, asq, c0i

        vt0, vt1, vtc, ay0, aysq, cy0i = run_pass(
            ybufs, y_start, y_wait, yg, (z, z, z, z, z, zi))

        inv_l = jnp.float32(1.0 / L)
        outs = (vt0, vt1, vtc,
                ax0, axsq - ax0, cx0i.astype(jnp.float32) * inv_l,
                ay0, aysq - ay0, cy0i.astype(jnp.float32) * inv_l)
        for k, v in enumerate(outs):
            ostage[pl.ds(k * L, L)] = v
        pltpu.sync_copy(ostage, out_hbm.at[wid])

    return acb_sc


_ACB = _make_kernel()


def kernel(reconstructed_image, target_image):
    cat = jnp.concatenate([reconstructed_image.reshape(NROWS, W),
                           target_image.reshape(NROWS, W)], axis=0)
    parts = _ACB(cat)
    p = parts.sum(axis=0).reshape(NACC, L).sum(axis=1)

    def term(s0, s1, c0, total):
        n1 = total - c0
        zl = jnp.where(c0 > 0, s0 / jnp.maximum(c0, 1.0), 0.0)
        nl = jnp.where(n1 > 0, s1 / jnp.maximum(n1, 1.0), 0.0)
        return zl + nl

    vt = term(p[0], p[1], p[2], float(B * H * W))
    vx = term(p[3], p[4], p[5], float(B * W * TS))
    vy = term(p[6], p[7], p[8] - PHANTOM, float(B * H * TS))
    return vt + vx + vy


# speedup vs baseline: 1.0521x; 1.0521x over previous
import functools

import jax
import jax.numpy as jnp
from jax import lax
from jax.experimental import pallas as pl
from jax.experimental.pallas import tpu as pltpu
from jax.experimental.pallas import tpu_sc as plsc

L = 16
NC = 2
NS = 16
NW = NC * NS
B, H, W = 16, 512, 512
NROWS = B * H
TS = 1000
ROWW = 1024
TRASH_X = TS
TRASH_Y = 1008
HOLW = L * ROWW
GROUPS = NROWS // L
G_PER_W = GROUPS // NW
NACC = 9
UNROLL = 8
CHX = 8
CHY = 4
PHANTOM = 8.0 * NROWS


def _q16(x, trash):
    q0 = (x * jnp.float32(TS)).astype(jnp.int32) - 1
    q = jnp.where(q0 < 0, q0 + TS, q0)
    return jnp.where(x == 0.0, trash, q)


def _make_kernel():
    mesh = plsc.VectorSubcoreMesh(core_axis_name="c", subcore_axis_name="s")

    @functools.partial(
        pl.kernel,
        out_type=jax.ShapeDtypeStruct((NW, NACC * L), jnp.float32),
        mesh=mesh,
        compiler_params=pltpu.CompilerParams(
            use_tc_tiling_on_sc=False, needs_layout_passes=False),
        scratch_types=[
            pltpu.VMEM((H, L), jnp.float32),
            pltpu.VMEM((H, L), jnp.float32),
            pltpu.VMEM((H, L), jnp.float32),
            pltpu.VMEM((H, L), jnp.float32),
            pltpu.VMEM((L, W), jnp.float32),
            pltpu.VMEM((L, W), jnp.float32),
            pltpu.VMEM((L, W), jnp.float32),
            pltpu.VMEM((L, W), jnp.float32),
            pltpu.VMEM((HOLW,), jnp.float32),
            pltpu.VMEM((HOLW,), jnp.float32),
            pltpu.VMEM((NACC * L,), jnp.float32),
            pltpu.SemaphoreType.DMA,
            pltpu.SemaphoreType.DMA,
            pltpu.SemaphoreType.DMA,
            pltpu.SemaphoreType.DMA,
            pltpu.SemaphoreType.DMA,
            pltpu.SemaphoreType.DMA,
            pltpu.SemaphoreType.DMA,
            pltpu.SemaphoreType.DMA,
        ],
    )
    def acb_sc(rec_hbm, tgt_hbm, out_hbm,
               xr0, xt0, xr1, xt1, yr0, yt0, yr1, yt1, hr, ht, ostage,
               sxr0, sxt0, sxr1, sxt1, syr0, syt0, syr1, syt1):
        xbufs = ((xr0, xt0, sxr0, sxt0), (xr1, xt1, sxr1, sxt1))
        ybufs = ((yr0, yt0, syr0, syt0), (yr1, yt1, syr1, syt1))
        wid = lax.axis_index("s") * NC + lax.axis_index("c")
        iot = lax.iota(jnp.int32, L)
        iotf = iot.astype(jnp.float32)
        perm = jnp.minimum(iot + 1, L - 1)
        last_lane = iot == (L - 1)
        zero16 = jnp.zeros((L,), jnp.float32)

        def zinit(k, _):
            hr[pl.ds(k * L, L)] = zero16
            ht[pl.ds(k * L, L)] = zero16
            return 0
        lax.fori_loop(0, HOLW // L, zinit, 0, unroll=UNROLL)

        def stats(r, t, a0, a1, c0):
            d = r - t
            dd = d * d
            z = t == 0.0
            return (a0 + jnp.where(z, dd, 0.0),
                    a1 + jnp.where(z, 0.0, dd),
                    c0 + jnp.where(z, 1.0, 0.0))

        def scan_chunk(off, accs):
            a0, asq, c0i = accs
            r = hr[pl.ds(off, L)]
            t = ht[pl.ds(off, L)]
            hr[pl.ds(off, L)] = zero16
            ht[pl.ds(off, L)] = zero16
            d = r - t
            dd = d * d
            z = t == 0.0
            return (a0 + jnp.where(z, dd, 0.0), asq + dd,
                    c0i + plsc.all_reduce_population_count(z))

        def x_start(i, buf):
            rbuf, tbuf, rsem, tsem = buf
            g = wid * G_PER_W + i
            b = g // (W // L)
            m0 = (g % (W // L)) * L
            src = lambda h: h.at[pl.ds(b * H, H), pl.ds(m0, L)]
            pltpu.async_copy(src(rec_hbm), rbuf, rsem)
            pltpu.async_copy(src(tgt_hbm), tbuf, tsem)

        def x_wait(buf):
            rbuf, tbuf, rsem, tsem = buf
            dummy = lambda h: h.at[pl.ds(0, H), pl.ds(0, L)]
            pltpu.make_async_copy(dummy(rec_hbm), rbuf, rsem).wait()
            pltpu.make_async_copy(dummy(tgt_hbm), tbuf, tsem).wait()

        def x_group(buf):
            rbuf, tbuf = buf[0], buf[1]

            def chunk(i, _):
                idxs, vals = [], []
                for k in range(CHX):
                    s = i * CHX + k
                    sb = jnp.broadcast_to(s, (L,))
                    r = plsc.load_gather(rbuf, [sb, iot])
                    t = plsc.load_gather(tbuf, [sb, iot])
                    idxs.append((_q16(r, TRASH_X) * L + iot,
                                 _q16(t, TRASH_X) * L + iot))
                    vals.append(jnp.broadcast_to(s.astype(jnp.float32), (L,)))
                for (qr, qt), sf in zip(idxs, vals):
                    plsc.store_scatter(hr, [qr], sf)
                    plsc.store_scatter(ht, [qt], sf)
                return 0

            lax.fori_loop(0, H // CHX, chunk, 0)

        def x_scan(accs):
            def body(k, accs):
                return scan_chunk(k * L, accs)
            accs = lax.fori_loop(0, TS, body, accs, unroll=UNROLL)
            hr[pl.ds(TRASH_X * L, L)] = zero16
            ht[pl.ds(TRASH_X * L, L)] = zero16
            return accs

        def y_start(i, buf):
            rbuf, tbuf, rsem, tsem = buf
            g = wid * G_PER_W + i
            src = lambda h: h.at[pl.ds(g * L, L), :]
            pltpu.async_copy(src(rec_hbm), rbuf, rsem)
            pltpu.async_copy(src(tgt_hbm), tbuf, tsem)

        def y_wait(buf):
            rbuf, tbuf, rsem, tsem = buf
            dummy = lambda h: h.at[pl.ds(0, L), :]
            pltpu.make_async_copy(dummy(rec_hbm), rbuf, rsem).wait()
            pltpu.make_async_copy(dummy(tgt_hbm), tbuf, tsem).wait()

        def winners(x, cf, trash):
            q = _q16(x, trash)
            key = (q << 4) | iot
            sk, sv = plsc.sort_key_val(key, cf)
            qs = lax.shift_right_logical(sk, 4)
            nxt = qs.at[perm].get(mode="promise_in_bounds")
            keep = (qs != nxt) | last_lane
            return qs, sv, keep

        def y_group(buf, accs):
            rbuf, tbuf = buf[0], buf[1]

            def row(l, accs):
                lb = jnp.broadcast_to(l, (L,))
                hb = jnp.broadcast_to(l * ROWW, (L,))

                def chunk(jc, accs):
                    pend = []
                    for k in range(CHY):
                        j = jc * CHY + k
                        cb = jnp.broadcast_to(j * L, (L,)) + iot
                        r = plsc.load_gather(rbuf, [lb, cb])
                        t = plsc.load_gather(tbuf, [lb, cb])
                        accs = stats(r, t, *accs)
                        cf = cb.astype(jnp.float32)
                        pend.append(winners(r, cf, TRASH_Y)
                                    + winners(t, cf, TRASH_Y))
                    for qr, vr, kr, qt, vt_, kt_ in pend:
                        plsc.store_scatter(hr, [hb + qr], vr, mask=kr)
                        plsc.store_scatter(ht, [hb + qt], vt_, mask=kt_)
                    return accs

                return lax.fori_loop(0, W // L // CHY, chunk, accs)

            return lax.fori_loop(0, L, row, accs)

        def y_scan(accs):
            def row(l, accs):
                base = l * ROWW

                def body(k, accs):
                    return scan_chunk(base + k * L, accs)
                accs = lax.fori_loop(0, 63, body, accs, unroll=UNROLL)
                hr[pl.ds(base + TRASH_Y, L)] = zero16
                ht[pl.ds(base + TRASH_Y, L)] = zero16
                return accs

            return lax.fori_loop(0, L, row, accs)

        def run_pass(bufs, start, wait, group, accs):
            start(0, bufs[0])

            def body(k, accs):
                for parity in (0, 1):
                    i = 2 * k + parity
                    wait(bufs[parity])
                    start(jnp.minimum(i + 1, G_PER_W - 1), bufs[1 - parity])
                    accs = group(bufs[parity], accs)
                return accs

            accs = lax.fori_loop(0, G_PER_W // 2, body, accs)
            wait(bufs[0])
            return accs

        z = zero16
        zi = jnp.zeros((L,), jnp.int32)

        def xg(buf, accs):
            x_group(buf)
            return x_scan(accs)

        ax0, axsq, cx0i = run_pass(xbufs, x_start, x_wait, xg, (z, z, zi))

        def yg(buf, accs):
            vt0, vt1, vtc, a0, asq, c0i = accs
            vt0, vt1, vtc = y_group(buf, (vt0, vt1, vtc))
            a0, asq, c0i = y_scan((a0, asq, c0i))
            return vt0, vt1, vtc, a0, asq, c0i

        vt0, vt1, vtc, ay0, aysq, cy0i = run_pass(
            ybufs, y_start, y_wait, yg, (z, z, z, z, z, zi))

        inv_l = jnp.float32(1.0 / L)
        outs = (vt0, vt1, vtc,
                ax0, axsq - ax0, cx0i.astype(jnp.float32) * inv_l,
                ay0, aysq - ay0, cy0i.astype(jnp.float32) * inv_l)
        for k, v in enumerate(outs):
            ostage[pl.ds(k * L, L)] = v
        pltpu.sync_copy(ostage, out_hbm.at[wid])

    return acb_sc


_ACB = _make_kernel()


def kernel(reconstructed_image, target_image):
    rec = reconstructed_image.reshape(NROWS, W)
    tgt = target_image.reshape(NROWS, W)
    parts = _ACB(rec, tgt)
    p = parts.sum(axis=0).reshape(NACC, L).sum(axis=1)

    def term(s0, s1, c0, total):
        n1 = total - c0
        zl = jnp.where(c0 > 0, s0 / jnp.maximum(c0, 1.0), 0.0)
        nl = jnp.where(n1 > 0, s1 / jnp.maximum(n1, 1.0), 0.0)
        return zl + nl

    vt = term(p[0], p[1], p[2], float(B * H * W))
    vx = term(p[3], p[4], p[5], float(B * W * TS))
    vy = term(p[6], p[7], p[8] - PHANTOM, float(B * H * TS))
    return vt + vx + vy


# speedup vs baseline: 1.2032x; 1.1436x over previous
import functools

import jax
import jax.numpy as jnp
from jax import lax
from jax.experimental import pallas as pl
from jax.experimental.pallas import tpu as pltpu
from jax.experimental.pallas import tpu_sc as plsc

L = 16
NC = 2
NS = 16
NW = NC * NS
B, H, W = 16, 512, 512
NROWS = B * H
TS = 1000
ROWW = 1024
TRASH_X = TS
TRASH_Y = 1008
HOLW = L * ROWW
GROUPS = NROWS // L
G_PER_W = GROUPS // NW
NACC = 9
UNROLL = 8
CHX = 8
CHY = 4
PHANTOM = 8.0 * NROWS


def _q16(x, trash):
    q0 = (x * jnp.float32(TS)).astype(jnp.int32) - 1
    q = jnp.where(q0 < 0, q0 + TS, q0)
    return jnp.where(x == 0.0, trash, q)


def _make_kernel():
    mesh = plsc.VectorSubcoreMesh(core_axis_name="c", subcore_axis_name="s")

    @functools.partial(
        pl.kernel,
        out_type=jax.ShapeDtypeStruct((NW, NACC * L), jnp.float32),
        mesh=mesh,
        compiler_params=pltpu.CompilerParams(
            use_tc_tiling_on_sc=False, needs_layout_passes=False),
        scratch_types=[
            pltpu.VMEM((H, L), jnp.float32),
            pltpu.VMEM((H, L), jnp.float32),
            pltpu.VMEM((H, L), jnp.float32),
            pltpu.VMEM((H, L), jnp.float32),
            pltpu.VMEM((L, W), jnp.float32),
            pltpu.VMEM((L, W), jnp.float32),
            pltpu.VMEM((L, W), jnp.float32),
            pltpu.VMEM((L, W), jnp.float32),
            pltpu.VMEM((HOLW,), jnp.float32),
            pltpu.VMEM((HOLW,), jnp.float32),
            pltpu.VMEM((NACC * L,), jnp.float32),
            pltpu.SemaphoreType.DMA,
            pltpu.SemaphoreType.DMA,
            pltpu.SemaphoreType.DMA,
            pltpu.SemaphoreType.DMA,
            pltpu.SemaphoreType.DMA,
            pltpu.SemaphoreType.DMA,
            pltpu.SemaphoreType.DMA,
            pltpu.SemaphoreType.DMA,
        ],
    )
    def acb_sc(rec_hbm, tgt_hbm, out_hbm,
               xr0, xt0, xr1, xt1, yr0, yt0, yr1, yt1, hr, ht, ostage,
               sxr0, sxt0, sxr1, sxt1, syr0, syt0, syr1, syt1):
        xbufs = ((xr0, xt0, sxr0, sxt0), (xr1, xt1, sxr1, sxt1))
        ybufs = ((yr0, yt0, syr0, syt0), (yr1, yt1, syr1, syt1))
        wid = lax.axis_index("s") * NC + lax.axis_index("c")
        iot = lax.iota(jnp.int32, L)
        iotf = iot.astype(jnp.float32)
        perm = jnp.minimum(iot + 1, L - 1)
        last_lane = iot == (L - 1)
        zero16 = jnp.zeros((L,), jnp.float32)

        def zinit(k, _):
            hr[pl.ds(k * L, L)] = zero16
            ht[pl.ds(k * L, L)] = zero16
            return 0
        lax.fori_loop(0, HOLW // L, zinit, 0, unroll=UNROLL)

        def stats(r, t, a0, a1, c0):
            d = r - t
            dd = d * d
            z = t == 0.0
            return (a0 + jnp.where(z, dd, 0.0),
                    a1 + jnp.where(z, 0.0, dd),
                    c0 + jnp.where(z, 1.0, 0.0))

        def scan_chunk(off, accs):
            a0, asq, c0i = accs
            r = hr[pl.ds(off, L)]
            t = ht[pl.ds(off, L)]
            hr[pl.ds(off, L)] = zero16
            ht[pl.ds(off, L)] = zero16
            d = r - t
            dd = d * d
            z = t == 0.0
            return (a0 + jnp.where(z, dd, 0.0), asq + dd,
                    c0i + plsc.all_reduce_population_count(z))

        def x_start(i, buf):
            rbuf, tbuf, rsem, tsem = buf
            g = wid * G_PER_W + i
            b = g // (W // L)
            m0 = (g % (W // L)) * L
            src = lambda h: h.at[pl.ds(b * H, H), pl.ds(m0, L)]
            pltpu.async_copy(src(rec_hbm), rbuf, rsem)
            pltpu.async_copy(src(tgt_hbm), tbuf, tsem)

        def x_wait(buf):
            rbuf, tbuf, rsem, tsem = buf
            dummy = lambda h: h.at[pl.ds(0, H), pl.ds(0, L)]
            pltpu.make_async_copy(dummy(rec_hbm), rbuf, rsem).wait()
            pltpu.make_async_copy(dummy(tgt_hbm), tbuf, tsem).wait()

        def x_group(buf):
            rbuf, tbuf = buf[0], buf[1]

            def chunk(i, _):
                idxs, vals = [], []
                for k in range(CHX):
                    s = i * CHX + k
                    sb = jnp.broadcast_to(s, (L,))
                    r = plsc.load_gather(rbuf, [sb, iot])
                    t = plsc.load_gather(tbuf, [sb, iot])
                    idxs.append((_q16(r, TRASH_X) * L + iot,
                                 _q16(t, TRASH_X) * L + iot))
                    vals.append(jnp.broadcast_to(s.astype(jnp.float32), (L,)))
                for (qr, qt), sf in zip(idxs, vals):
                    plsc.store_scatter(hr, [qr], sf)
                    plsc.store_scatter(ht, [qt], sf)
                return 0

            lax.fori_loop(0, H // CHX, chunk, 0)

        def x_scan(accs):
            def body(k, accs):
                return scan_chunk(k * L, accs)
            accs = lax.fori_loop(0, TS, body, accs, unroll=UNROLL)
            hr[pl.ds(TRASH_X * L, L)] = zero16
            ht[pl.ds(TRASH_X * L, L)] = zero16
            return accs

        def y_start(i, buf):
            rbuf, tbuf, rsem, tsem = buf
            g = wid * G_PER_W + i
            src = lambda h: h.at[pl.ds(g * L, L), :]
            pltpu.async_copy(src(rec_hbm), rbuf, rsem)
            pltpu.async_copy(src(tgt_hbm), tbuf, tsem)

        def y_wait(buf):
            rbuf, tbuf, rsem, tsem = buf
            dummy = lambda h: h.at[pl.ds(0, L), :]
            pltpu.make_async_copy(dummy(rec_hbm), rbuf, rsem).wait()
            pltpu.make_async_copy(dummy(tgt_hbm), tbuf, tsem).wait()

        def winners(x, cf, trash):
            q = _q16(x, trash)
            key = (q << 4) | iot
            sk, sv = plsc.sort_key_val(key, cf)
            qs = lax.shift_right_logical(sk, 4)
            nxt = qs.at[perm].get(mode="promise_in_bounds")
            keep = (qs != nxt) | last_lane
            return qs, sv, keep

        def y_group(buf, accs):
            rbuf, tbuf = buf[0], buf[1]

            def row(l, accs):
                lb = jnp.broadcast_to(l, (L,))
                hb = jnp.broadcast_to(l * ROWW, (L,))

                def chunk(jc, accs):
                    pend = []
                    for k in range(CHY):
                        j = jc * CHY + k
                        cb = jnp.broadcast_to(j * L, (L,)) + iot
                        r = plsc.load_gather(rbuf, [lb, cb])
                        t = plsc.load_gather(tbuf, [lb, cb])
                        accs = stats(r, t, *accs)
                        cf = cb.astype(jnp.float32)
                        pend.append((_q16(r, TRASH_Y), _q16(t, TRASH_Y), cf))
                    for qr, qt, cf in pend:
                        plsc.store_scatter(hr, [hb + qr], cf)
                        plsc.store_scatter(ht, [hb + qt], cf)
                    return accs

                return lax.fori_loop(0, W // L // CHY, chunk, accs)

            return lax.fori_loop(0, L, row, accs)

        def y_scan(accs):
            def row(l, accs):
                base = l * ROWW

                def body(k, accs):
                    return scan_chunk(base + k * L, accs)
                accs = lax.fori_loop(0, 63, body, accs, unroll=UNROLL)
                hr[pl.ds(base + TRASH_Y, L)] = zero16
                ht[pl.ds(base + TRASH_Y, L)] = zero16
                return accs

            return lax.fori_loop(0, L, row, accs)

        def run_pass(bufs, start, wait, group, accs):
            start(0, bufs[0])

            def body(k, accs):
                for parity in (0, 1):
                    i = 2 * k + parity
                    wait(bufs[parity])
                    start(jnp.minimum(i + 1, G_PER_W - 1), bufs[1 - parity])
                    accs = group(bufs[parity], accs)
                return accs

            accs = lax.fori_loop(0, G_PER_W // 2, body, accs)
            wait(bufs[0])
            return accs

        z = zero16
        zi = jnp.zeros((L,), jnp.int32)

        def xg(buf, accs):
            x_group(buf)
            return x_scan(accs)

        ax0, axsq, cx0i = run_pass(xbufs, x_start, x_wait, xg, (z, z, zi))

        def yg(buf, accs):
            vt0, vt1, vtc, a0, asq, c0i = accs
            vt0, vt1, vtc = y_group(buf, (vt0, vt1, vtc))
            a0, asq, c0i = y_scan((a0, asq, c0i))
            return vt0, vt1, vtc, a0, asq, c0i

        vt0, vt1, vtc, ay0, aysq, cy0i = run_pass(
            ybufs, y_start, y_wait, yg, (z, z, z, z, z, zi))

        inv_l = jnp.float32(1.0 / L)
        outs = (vt0, vt1, vtc,
                ax0, axsq - ax0, cx0i.astype(jnp.float32) * inv_l,
                ay0, aysq - ay0, cy0i.astype(jnp.float32) * inv_l)
        for k, v in enumerate(outs):
            ostage[pl.ds(k * L, L)] = v
        pltpu.sync_copy(ostage, out_hbm.at[wid])

    return acb_sc


_ACB = _make_kernel()


def kernel(reconstructed_image, target_image):
    rec = reconstructed_image.reshape(NROWS, W)
    tgt = target_image.reshape(NROWS, W)
    parts = _ACB(rec, tgt)
    p = parts.sum(axis=0).reshape(NACC, L).sum(axis=1)

    def term(s0, s1, c0, total):
        n1 = total - c0
        zl = jnp.where(c0 > 0, s0 / jnp.maximum(c0, 1.0), 0.0)
        nl = jnp.where(n1 > 0, s1 / jnp.maximum(n1, 1.0), 0.0)
        return zl + nl

    vt = term(p[0], p[1], p[2], float(B * H * W))
    vx = term(p[3], p[4], p[5], float(B * W * TS))
    vy = term(p[6], p[7], p[8] - PHANTOM, float(B * H * TS))
    return vt + vx + vy


# speedup vs baseline: 1.2248x; 1.0180x over previous
import functools

import jax
import jax.numpy as jnp
from jax import lax
from jax.experimental import pallas as pl
from jax.experimental.pallas import tpu as pltpu
from jax.experimental.pallas import tpu_sc as plsc

L = 16
NC = 2
NS = 16
NW = NC * NS
B, H, W = 16, 512, 512
NROWS = B * H
TS = 1000
ROWW = 1024
TRASH_X = TS
TRASH_Y = 1008
HOLW = L * ROWW
GROUPS = NROWS // L
G_PER_W = GROUPS // NW
NACC = 9
UNROLL = 8
CHX = 8
CHY = 8
PHANTOM = 8.0 * NROWS


def _q16(x, trash):
    q0 = (x * jnp.float32(TS)).astype(jnp.int32) - 1
    q = jnp.where(q0 < 0, q0 + TS, q0)
    return jnp.where(x == 0.0, trash, q)


def _make_kernel():
    mesh = plsc.VectorSubcoreMesh(core_axis_name="c", subcore_axis_name="s")

    @functools.partial(
        pl.kernel,
        out_type=jax.ShapeDtypeStruct((NW, NACC * L), jnp.float32),
        mesh=mesh,
        compiler_params=pltpu.CompilerParams(
            use_tc_tiling_on_sc=False, needs_layout_passes=False),
        scratch_types=[
            pltpu.VMEM((H, L), jnp.float32),
            pltpu.VMEM((H, L), jnp.float32),
            pltpu.VMEM((H, L), jnp.float32),
            pltpu.VMEM((H, L), jnp.float32),
            pltpu.VMEM((L, W), jnp.float32),
            pltpu.VMEM((L, W), jnp.float32),
            pltpu.VMEM((L, W), jnp.float32),
            pltpu.VMEM((L, W), jnp.float32),
            pltpu.VMEM((HOLW,), jnp.float32),
            pltpu.VMEM((HOLW,), jnp.float32),
            pltpu.VMEM((NACC * L,), jnp.float32),
            pltpu.SemaphoreType.DMA,
            pltpu.SemaphoreType.DMA,
            pltpu.SemaphoreType.DMA,
            pltpu.SemaphoreType.DMA,
            pltpu.SemaphoreType.DMA,
            pltpu.SemaphoreType.DMA,
            pltpu.SemaphoreType.DMA,
            pltpu.SemaphoreType.DMA,
        ],
    )
    def acb_sc(rec_hbm, tgt_hbm, out_hbm,
               xr0, xt0, xr1, xt1, yr0, yt0, yr1, yt1, hr, ht, ostage,
               sxr0, sxt0, sxr1, sxt1, syr0, syt0, syr1, syt1):
        xbufs = ((xr0, xt0, sxr0, sxt0), (xr1, xt1, sxr1, sxt1))
        ybufs = ((yr0, yt0, syr0, syt0), (yr1, yt1, syr1, syt1))
        wid = lax.axis_index("s") * NC + lax.axis_index("c")
        iot = lax.iota(jnp.int32, L)
        iotf = iot.astype(jnp.float32)
        perm = jnp.minimum(iot + 1, L - 1)
        last_lane = iot == (L - 1)
        zero16 = jnp.zeros((L,), jnp.float32)

        def zinit(k, _):
            hr[pl.ds(k * L, L)] = zero16
            ht[pl.ds(k * L, L)] = zero16
            return 0
        lax.fori_loop(0, HOLW // L, zinit, 0, unroll=UNROLL)

        def stats(r, t, a0, a1, c0):
            d = r - t
            dd = d * d
            z = t == 0.0
            return (a0 + jnp.where(z, dd, 0.0),
                    a1 + jnp.where(z, 0.0, dd),
                    c0 + jnp.where(z, 1.0, 0.0))

        def scan_chunk(off, accs):
            a0, asq, c0i = accs
            r = hr[pl.ds(off, L)]
            t = ht[pl.ds(off, L)]
            hr[pl.ds(off, L)] = zero16
            ht[pl.ds(off, L)] = zero16
            d = r - t
            dd = d * d
            z = t == 0.0
            return (a0 + jnp.where(z, dd, 0.0), asq + dd,
                    c0i + plsc.all_reduce_population_count(z))

        def x_start(i, buf):
            rbuf, tbuf, rsem, tsem = buf
            g = wid * G_PER_W + i
            b = g // (W // L)
            m0 = (g % (W // L)) * L
            src = lambda h: h.at[pl.ds(b * H, H), pl.ds(m0, L)]
            pltpu.async_copy(src(rec_hbm), rbuf, rsem)
            pltpu.async_copy(src(tgt_hbm), tbuf, tsem)

        def x_wait(buf):
            rbuf, tbuf, rsem, tsem = buf
            dummy = lambda h: h.at[pl.ds(0, H), pl.ds(0, L)]
            pltpu.make_async_copy(dummy(rec_hbm), rbuf, rsem).wait()
            pltpu.make_async_copy(dummy(tgt_hbm), tbuf, tsem).wait()

        def x_group(buf):
            rbuf, tbuf = buf[0], buf[1]

            def chunk(i, _):
                idxs, vals = [], []
                for k in range(CHX):
                    s = i * CHX + k
                    sb = jnp.broadcast_to(s, (L,))
                    r = plsc.load_gather(rbuf, [sb, iot])
                    t = plsc.load_gather(tbuf, [sb, iot])
                    idxs.append((_q16(r, TRASH_X) * L + iot,
                                 _q16(t, TRASH_X) * L + iot))
                    vals.append(jnp.broadcast_to(s.astype(jnp.float32), (L,)))
                for (qr, qt), sf in zip(idxs, vals):
                    plsc.store_scatter(hr, [qr], sf)
                    plsc.store_scatter(ht, [qt], sf)
                return 0

            lax.fori_loop(0, H // CHX, chunk, 0)

        def x_scan(accs):
            def body(k, accs):
                return scan_chunk(k * L, accs)
            accs = lax.fori_loop(0, TS, body, accs, unroll=UNROLL)
            hr[pl.ds(TRASH_X * L, L)] = zero16
            ht[pl.ds(TRASH_X * L, L)] = zero16
            return accs

        def y_start(i, buf):
            rbuf, tbuf, rsem, tsem = buf
            g = wid * G_PER_W + i
            src = lambda h: h.at[pl.ds(g * L, L), :]
            pltpu.async_copy(src(rec_hbm), rbuf, rsem)
            pltpu.async_copy(src(tgt_hbm), tbuf, tsem)

        def y_wait(buf):
            rbuf, tbuf, rsem, tsem = buf
            dummy = lambda h: h.at[pl.ds(0, L), :]
            pltpu.make_async_copy(dummy(rec_hbm), rbuf, rsem).wait()
            pltpu.make_async_copy(dummy(tgt_hbm), tbuf, tsem).wait()

        def winners(x, cf, trash):
            q = _q16(x, trash)
            key = (q << 4) | iot
            sk, sv = plsc.sort_key_val(key, cf)
            qs = lax.shift_right_logical(sk, 4)
            nxt = qs.at[perm].get(mode="promise_in_bounds")
            keep = (qs != nxt) | last_lane
            return qs, sv, keep

        def y_group(buf, accs):
            rbuf, tbuf = buf[0], buf[1]

            def row(l, accs):
                lb = jnp.broadcast_to(l, (L,))
                hb = jnp.broadcast_to(l * ROWW, (L,))

                def chunk(jc, accs):
                    pend = []
                    for k in range(CHY):
                        j = jc * CHY + k
                        cb = jnp.broadcast_to(j * L, (L,)) + iot
                        r = plsc.load_gather(rbuf, [lb, cb])
                        t = plsc.load_gather(tbuf, [lb, cb])
                        accs = stats(r, t, *accs)
                        cf = cb.astype(jnp.float32)
                        pend.append((_q16(r, TRASH_Y), _q16(t, TRASH_Y), cf))
                    for qr, qt, cf in pend:
                        plsc.store_scatter(hr, [hb + qr], cf)
                        plsc.store_scatter(ht, [hb + qt], cf)
                    return accs

                return lax.fori_loop(0, W // L // CHY, chunk, accs)

            return lax.fori_loop(0, L, row, accs)

        def y_scan(accs):
            def row(l, accs):
                base = l * ROWW

                def body(k, accs):
                    return scan_chunk(base + k * L, accs)
                accs = lax.fori_loop(0, 63, body, accs, unroll=UNROLL)
                hr[pl.ds(base + TRASH_Y, L)] = zero16
                ht[pl.ds(base + TRASH_Y, L)] = zero16
                return accs

            return lax.fori_loop(0, L, row, accs)

        def run_pass(bufs, start, wait, group, accs):
            start(0, bufs[0])

            def body(k, accs):
                for parity in (0, 1):
                    i = 2 * k + parity
                    wait(bufs[parity])
                    start(jnp.minimum(i + 1, G_PER_W - 1), bufs[1 - parity])
                    accs = group(bufs[parity], accs)
                return accs

            accs = lax.fori_loop(0, G_PER_W // 2, body, accs)
            wait(bufs[0])
            return accs

        z = zero16
        zi = jnp.zeros((L,), jnp.int32)

        def xg(buf, accs):
            x_group(buf)
            return x_scan(accs)

        ax0, axsq, cx0i = run_pass(xbufs, x_start, x_wait, xg, (z, z, zi))

        def yg(buf, accs):
            vt0, vt1, vtc, a0, asq, c0i = accs
            vt0, vt1, vtc = y_group(buf, (vt0, vt1, vtc))
            a0, asq, c0i = y_scan((a0, asq, c0i))
            return vt0, vt1, vtc, a0, asq, c0i

        vt0, vt1, vtc, ay0, aysq, cy0i = run_pass(
            ybufs, y_start, y_wait, yg, (z, z, z, z, z, zi))

        inv_l = jnp.float32(1.0 / L)
        outs = (vt0, vt1, vtc,
                ax0, axsq - ax0, cx0i.astype(jnp.float32) * inv_l,
                ay0, aysq - ay0, cy0i.astype(jnp.float32) * inv_l)
        for k, v in enumerate(outs):
            ostage[pl.ds(k * L, L)] = v
        pltpu.sync_copy(ostage, out_hbm.at[wid])

    return acb_sc


_ACB = _make_kernel()


def kernel(reconstructed_image, target_image):
    rec = reconstructed_image.reshape(NROWS, W)
    tgt = target_image.reshape(NROWS, W)
    parts = _ACB(rec, tgt)
    p = parts.sum(axis=0).reshape(NACC, L).sum(axis=1)

    def term(s0, s1, c0, total):
        n1 = total - c0
        zl = jnp.where(c0 > 0, s0 / jnp.maximum(c0, 1.0), 0.0)
        nl = jnp.where(n1 > 0, s1 / jnp.maximum(n1, 1.0), 0.0)
        return zl + nl

    vt = term(p[0], p[1], p[2], float(B * H * W))
    vx = term(p[3], p[4], p[5], float(B * W * TS))
    vy = term(p[6], p[7], p[8] - PHANTOM, float(B * H * TS))
    return vt + vx + vy


# speedup vs baseline: 1.2675x; 1.0348x over previous
import functools

import jax
import jax.numpy as jnp
from jax import lax
from jax.experimental import pallas as pl
from jax.experimental.pallas import tpu as pltpu
from jax.experimental.pallas import tpu_sc as plsc

L = 16
NC = 2
NS = 16
NW = NC * NS
B, H, W = 16, 512, 512
NROWS = B * H
TS = 1000
ROWW = 1024
TRASH_X = TS
TRASH_Y = 1008
HOLW = L * ROWW
GROUPS = NROWS // L
G_PER_W = GROUPS // NW
NACC = 9
UNROLL = 16
CHX = 16
CHY = 16
PHANTOM = 8.0 * NROWS


def _q16(x, trash):
    q0 = (x * jnp.float32(TS)).astype(jnp.int32) - 1
    q = jnp.where(q0 < 0, q0 + TS, q0)
    return jnp.where(x == 0.0, trash, q)


def _make_kernel():
    mesh = plsc.VectorSubcoreMesh(core_axis_name="c", subcore_axis_name="s")

    @functools.partial(
        pl.kernel,
        out_type=jax.ShapeDtypeStruct((NW, NACC * L), jnp.float32),
        mesh=mesh,
        compiler_params=pltpu.CompilerParams(
            use_tc_tiling_on_sc=False, needs_layout_passes=False),
        scratch_types=[
            pltpu.VMEM((H, L), jnp.float32),
            pltpu.VMEM((H, L), jnp.float32),
            pltpu.VMEM((H, L), jnp.float32),
            pltpu.VMEM((H, L), jnp.float32),
            pltpu.VMEM((L, W), jnp.float32),
            pltpu.VMEM((L, W), jnp.float32),
            pltpu.VMEM((L, W), jnp.float32),
            pltpu.VMEM((L, W), jnp.float32),
            pltpu.VMEM((HOLW,), jnp.float32),
            pltpu.VMEM((HOLW,), jnp.float32),
            pltpu.VMEM((NACC * L,), jnp.float32),
            pltpu.SemaphoreType.DMA,
            pltpu.SemaphoreType.DMA,
            pltpu.SemaphoreType.DMA,
            pltpu.SemaphoreType.DMA,
            pltpu.SemaphoreType.DMA,
            pltpu.SemaphoreType.DMA,
            pltpu.SemaphoreType.DMA,
            pltpu.SemaphoreType.DMA,
        ],
    )
    def acb_sc(rec_hbm, tgt_hbm, out_hbm,
               xr0, xt0, xr1, xt1, yr0, yt0, yr1, yt1, hr, ht, ostage,
               sxr0, sxt0, sxr1, sxt1, syr0, syt0, syr1, syt1):
        xbufs = ((xr0, xt0, sxr0, sxt0), (xr1, xt1, sxr1, sxt1))
        ybufs = ((yr0, yt0, syr0, syt0), (yr1, yt1, syr1, syt1))
        wid = lax.axis_index("s") * NC + lax.axis_index("c")
        iot = lax.iota(jnp.int32, L)
        zero16 = jnp.zeros((L,), jnp.float32)

        def zinit(k, _):
            hr[pl.ds(k * L, L)] = zero16
            ht[pl.ds(k * L, L)] = zero16
            return 0
        lax.fori_loop(0, HOLW // L, zinit, 0, unroll=UNROLL)

        def stats(r, t, a0, a1, c0):
            d = r - t
            dd = d * d
            z = t == 0.0
            return (a0 + jnp.where(z, dd, 0.0),
                    a1 + jnp.where(z, 0.0, dd),
                    c0 + jnp.where(z, 1.0, 0.0))

        def scan_chunk(off, accs):
            a0, asq, c0i = accs
            r = hr[pl.ds(off, L)]
            t = ht[pl.ds(off, L)]
            hr[pl.ds(off, L)] = zero16
            ht[pl.ds(off, L)] = zero16
            d = r - t
            dd = d * d
            z = t == 0.0
            return (a0 + jnp.where(z, dd, 0.0), asq + dd,
                    c0i + plsc.all_reduce_population_count(z))

        def x_start(i, buf):
            rbuf, tbuf, rsem, tsem = buf
            g = wid * G_PER_W + i
            b = g // (W // L)
            m0 = (g % (W // L)) * L
            src = lambda h: h.at[pl.ds(b * H, H), pl.ds(m0, L)]
            pltpu.async_copy(src(rec_hbm), rbuf, rsem)
            pltpu.async_copy(src(tgt_hbm), tbuf, tsem)

        def x_wait(buf):
            rbuf, tbuf, rsem, tsem = buf
            dummy = lambda h: h.at[pl.ds(0, H), pl.ds(0, L)]
            pltpu.make_async_copy(dummy(rec_hbm), rbuf, rsem).wait()
            pltpu.make_async_copy(dummy(tgt_hbm), tbuf, tsem).wait()

        def x_group(buf):
            rbuf, tbuf = buf[0], buf[1]

            def chunk(i, _):
                idxs, vals = [], []
                for k in range(CHX):
                    s = i * CHX + k
                    sb = jnp.broadcast_to(s, (L,))
                    r = plsc.load_gather(rbuf, [sb, iot])
                    t = plsc.load_gather(tbuf, [sb, iot])
                    idxs.append((_q16(r, TRASH_X) * L + iot,
                                 _q16(t, TRASH_X) * L + iot))
                    vals.append(jnp.broadcast_to(s.astype(jnp.float32), (L,)))
                for (qr, qt), sf in zip(idxs, vals):
                    plsc.store_scatter(hr, [qr], sf)
                    plsc.store_scatter(ht, [qt], sf)
                return 0

            lax.fori_loop(0, H // CHX, chunk, 0)

        def x_scan(accs):
            def body(k, accs):
                return scan_chunk(k * L, accs)
            accs = lax.fori_loop(0, TS, body, accs, unroll=UNROLL)
            hr[pl.ds(TRASH_X * L, L)] = zero16
            ht[pl.ds(TRASH_X * L, L)] = zero16
            return accs

        def y_start(i, buf):
            rbuf, tbuf, rsem, tsem = buf
            g = wid * G_PER_W + i
            src = lambda h: h.at[pl.ds(g * L, L), :]
            pltpu.async_copy(src(rec_hbm), rbuf, rsem)
            pltpu.async_copy(src(tgt_hbm), tbuf, tsem)

        def y_wait(buf):
            rbuf, tbuf, rsem, tsem = buf
            dummy = lambda h: h.at[pl.ds(0, L), :]
            pltpu.make_async_copy(dummy(rec_hbm), rbuf, rsem).wait()
            pltpu.make_async_copy(dummy(tgt_hbm), tbuf, tsem).wait()

        def y_group(buf, accs):
            rbuf, tbuf = buf[0], buf[1]

            def row(l, accs):
                lb = jnp.broadcast_to(l, (L,))
                hb = jnp.broadcast_to(l * ROWW, (L,))

                def chunk(jc, accs):
                    pend = []
                    for k in range(CHY):
                        j = jc * CHY + k
                        cb = jnp.broadcast_to(j * L, (L,)) + iot
                        r = plsc.load_gather(rbuf, [lb, cb])
                        t = plsc.load_gather(tbuf, [lb, cb])
                        accs = stats(r, t, *accs)
                        cf = cb.astype(jnp.float32)
                        pend.append((_q16(r, TRASH_Y), _q16(t, TRASH_Y), cf))
                    for qr, qt, cf in pend:
                        plsc.store_scatter(hr, [hb + qr], cf)
                        plsc.store_scatter(ht, [hb + qt], cf)
                    return accs

                return lax.fori_loop(0, W // L // CHY, chunk, accs)

            return lax.fori_loop(0, L, row, accs)

        def y_scan(accs):
            def row(l, accs):
                base = l * ROWW

                def body(k, accs):
                    return scan_chunk(base + k * L, accs)
                accs = lax.fori_loop(0, 63, body, accs, unroll=UNROLL)
                hr[pl.ds(base + TRASH_Y, L)] = zero16
                ht[pl.ds(base + TRASH_Y, L)] = zero16
                return accs

            return lax.fori_loop(0, L, row, accs)

        def run_pass(bufs, start, wait, group, accs):
            start(0, bufs[0])

            def body(k, accs):
                for parity in (0, 1):
                    i = 2 * k + parity
                    wait(bufs[parity])
                    start(jnp.minimum(i + 1, G_PER_W - 1), bufs[1 - parity])
                    accs = group(bufs[parity], accs)
                return accs

            accs = lax.fori_loop(0, G_PER_W // 2, body, accs)
            wait(bufs[0])
            return accs

        z = zero16
        zi = jnp.zeros((L,), jnp.int32)

        def xg(buf, accs):
            x_group(buf)
            return x_scan(accs)

        ax0, axsq, cx0i = run_pass(xbufs, x_start, x_wait, xg, (z, z, zi))

        def yg(buf, accs):
            vt0, vt1, vtc, a0, asq, c0i = accs
            vt0, vt1, vtc = y_group(buf, (vt0, vt1, vtc))
            a0, asq, c0i = y_scan((a0, asq, c0i))
            return vt0, vt1, vtc, a0, asq, c0i

        vt0, vt1, vtc, ay0, aysq, cy0i = run_pass(
            ybufs, y_start, y_wait, yg, (z, z, z, z, z, zi))

        inv_l = jnp.float32(1.0 / L)
        outs = (vt0, vt1, vtc,
                ax0, axsq - ax0, cx0i.astype(jnp.float32) * inv_l,
                ay0, aysq - ay0, cy0i.astype(jnp.float32) * inv_l)
        for k, v in enumerate(outs):
            ostage[pl.ds(k * L, L)] = v
        pltpu.sync_copy(ostage, out_hbm.at[wid])

    return acb_sc


_ACB = _make_kernel()


def kernel(reconstructed_image, target_image):
    rec = reconstructed_image.reshape(NROWS, W)
    tgt = target_image.reshape(NROWS, W)
    parts = _ACB(rec, tgt)
    p = parts.sum(axis=0).reshape(NACC, L).sum(axis=1)

    def term(s0, s1, c0, total):
        n1 = total - c0
        zl = jnp.where(c0 > 0, s0 / jnp.maximum(c0, 1.0), 0.0)
        nl = jnp.where(n1 > 0, s1 / jnp.maximum(n1, 1.0), 0.0)
        return zl + nl

    vt = term(p[0], p[1], p[2], float(B * H * W))
    vx = term(p[3], p[4], p[5], float(B * W * TS))
    vy = term(p[6], p[7], p[8] - PHANTOM, float(B * H * TS))
    return vt + vx + vy
